# Initial kernel scaffold; baseline (speedup 1.0000x reference)
import jax
import jax.numpy as jnp
from jax.experimental import pallas as pl


def kernel(mut_tokens, wt_tokens, hla_tokens, delta_tokens, scalars, embedding, W1, b1, W2, b2, W3, b3):
    raise NotImplementedError("write your pallas kernel here")


# speedup vs baseline: 105.5426x; 105.5426x over previous
import functools

import jax
import jax.numpy as jnp
from jax.experimental import pallas as pl
from jax.experimental.pallas import tpu as pltpu

EMBED_DIM = 16
HIDDEN_DIM = 32
VOCAB = 21
VPAD = 24
SEGS = ((0, 11), (11, 11), (22, 34), (56, 11))
TOK_TOTAL = 67
NB = 512


def _tc_body(toks_ref, scal_ref, bd_ref, w1t_ref, b1_ref, w2at_ref,
             w2bt_ref, b2_ref, w3t_ref, b3_ref, out_ref):
    toks = toks_ref[...]
    viota = jax.lax.broadcasted_iota(jnp.int32, (VPAD, NB), 0)
    slabs = []
    for seg_idx, (start, length) in enumerate(SEGS):
        slab = jnp.zeros((VPAD, NB), jnp.float32)
        for p in range(length):
            tokp = toks[start + p, :][None, :]
            slab = slab + jnp.where(viota == tokp, 1.0, 0.0)
        denom = jnp.maximum(float(length) - slab[0:1, :], 1.0)
        slabs.append(slab * (1.0 / denom))
    counts = jnp.concatenate(slabs, axis=0)
    pooled = jnp.dot(bd_ref[...], counts,
                     preferred_element_type=jnp.float32)
    sf = jnp.maximum(
        jnp.dot(w1t_ref[...], scal_ref[...],
                preferred_element_type=jnp.float32) + b1_ref[...], 0.0)
    h = jnp.maximum(
        jnp.dot(w2at_ref[...], pooled, preferred_element_type=jnp.float32)
        + jnp.dot(w2bt_ref[...], sf, preferred_element_type=jnp.float32)
        + b2_ref[...], 0.0)
    out_ref[...] = (jnp.dot(w3t_ref[...], h,
                            preferred_element_type=jnp.float32)
                    + b3_ref[...])


@jax.jit
def _tc_call(toks_t, scalars_t, bd, w1t, b1c, w2at, w2bt, b2c, w3t, b3c):
    batch = toks_t.shape[1]
    grid = (batch // NB,)
    return pl.pallas_call(
        _tc_body,
        grid=grid,
        in_specs=[
            pl.BlockSpec((TOK_TOTAL, NB), lambda j: (0, j)),
            pl.BlockSpec((10, NB), lambda j: (0, j)),
            pl.BlockSpec((4 * EMBED_DIM, 4 * VPAD), lambda j: (0, 0)),
            pl.BlockSpec((HIDDEN_DIM, 10), lambda j: (0, 0)),
            pl.BlockSpec((HIDDEN_DIM, 1), lambda j: (0, 0)),
            pl.BlockSpec((HIDDEN_DIM, 4 * EMBED_DIM), lambda j: (0, 0)),
            pl.BlockSpec((HIDDEN_DIM, HIDDEN_DIM), lambda j: (0, 0)),
            pl.BlockSpec((HIDDEN_DIM, 1), lambda j: (0, 0)),
            pl.BlockSpec((1, HIDDEN_DIM), lambda j: (0, 0)),
            pl.BlockSpec((1, 1), lambda j: (0, 0)),
        ],
        out_specs=pl.BlockSpec((1, NB), lambda j: (0, j)),
        out_shape=jax.ShapeDtypeStruct((1, batch), jnp.float32),
        compiler_params=pltpu.CompilerParams(
            dimension_semantics=("parallel",)),
    )(toks_t, scalars_t, bd, w1t, b1c, w2at, w2bt, b2c, w3t, b3c)


def kernel(mut_tokens, wt_tokens, hla_tokens, delta_tokens, scalars,
           embedding, W1, b1, W2, b2, W3, b3):
    toks_t = jnp.concatenate(
        [mut_tokens, wt_tokens, hla_tokens, delta_tokens],
        axis=1).astype(jnp.int32).T
    scalars_t = scalars.T
    et = embedding.T
    et = et.at[:, 0].set(0.0)
    etp = jnp.pad(et, ((0, 0), (0, VPAD - VOCAB)))
    bd = jnp.zeros((4 * EMBED_DIM, 4 * VPAD), jnp.float32)
    for s in range(4):
        bd = bd.at[s * EMBED_DIM:(s + 1) * EMBED_DIM,
                   s * VPAD:(s + 1) * VPAD].set(etp)
    out = _tc_call(toks_t, scalars_t, bd, W1.T, b1[:, None], W2[:64].T,
                   W2[64:].T, b2[:, None], W3.T, b3[:, None])
    return out[0]
